# baseline (speedup 1.0000x reference)
import functools

import jax
import jax.numpy as jnp
from jax import lax
from jax.experimental import pallas as pl
from jax.experimental.pallas import tpu as pltpu
from jax.experimental.pallas import tpu_sc as plsc

_N = 10000
_D = 128
_H1 = 32
_H2 = 16
_E = 160000

_CH = 125
_NT = 16
_EPT = _E // _NT
_NCH = _EPT // _CH
_RPT = 632
_NPAD = _RPT * _NT

_BM = 200
_BN = 10000


def _seg_sum_body(src_hbm, dst_hbm, tab_hbm, out_hbm,
                  sidx, didx, rows, zbuf, acc, sem):
    c = lax.axis_index("c")
    s = lax.axis_index("s")

    def zstep(i, carry):
        zbuf[i, pl.ds(0, 16)] = jnp.zeros((16,), jnp.float32)
        zbuf[i, pl.ds(16, 16)] = jnp.zeros((16,), jnp.float32)
        return carry

    lax.fori_loop(0, _RPT, zstep, 0)
    pltpu.sync_copy(zbuf, acc.at[pl.ds(s * _RPT, _RPT)])

    pltpu.sync_copy(src_hbm.at[c, pl.ds(s * _NCH, _NCH)], sidx)
    pltpu.sync_copy(dst_hbm.at[c, pl.ds(s * _NCH, _NCH)], didx)
    plsc.subcore_barrier()

    def estep(j, carry):
        pltpu.async_copy(tab_hbm.at[sidx.at[j]], rows, sem).wait()
        pltpu.sync_copy(rows, acc.at[didx.at[j]], add=True)
        return carry

    lax.fori_loop(0, _NCH, estep, 0)

    plsc.subcore_barrier()
    pltpu.sync_copy(acc.at[pl.ds(s * _RPT, _RPT)],
                    out_hbm.at[c, pl.ds(s * _RPT, _RPT)])


_seg_sum_kernel_cache = []


def _segment_sum_sc(src, dst, tab):
    if not _seg_sum_kernel_cache:
        mesh = plsc.VectorSubcoreMesh(core_axis_name="c", subcore_axis_name="s",
                                      num_cores=2, num_subcores=_NT)
        k = pl.kernel(
            _seg_sum_body,
            out_type=jax.ShapeDtypeStruct((2, _NPAD, _H1), jnp.float32),
            mesh=mesh,
            scratch_types=[
                pltpu.VMEM((_NCH, _CH), jnp.int32),
                pltpu.VMEM((_NCH, _CH), jnp.int32),
                pltpu.VMEM((_CH, _H1), jnp.float32),
                pltpu.VMEM((_RPT, _H1), jnp.float32),
                pltpu.VMEM_SHARED((_NPAD, _H1), jnp.float32),
                pltpu.SemaphoreType.DMA,
            ],
            compiler_params=pltpu.CompilerParams(use_tc_tiling_on_sc=False),
        )
        _seg_sum_kernel_cache.append(k)
    return _seg_sum_kernel_cache[0](src, dst, tab)


def _xw_body(x_ref, w_ref, o_ref):
    o_ref[0] = jnp.dot(x_ref[...], w_ref[0],
                       preferred_element_type=jnp.float32)


def _relu_noise_body(l1_ref, nz_ref, o_ref):
    o_ref[...] = jnp.maximum(l1_ref[...], 0.0) + nz_ref[...]


def _combine_body(s2_ref, w2_ref, wa_ref, z_ref):
    e0 = jnp.dot(s2_ref[0], w2_ref[0], preferred_element_type=jnp.float32)
    e1 = jnp.dot(s2_ref[1], w2_ref[1], preferred_element_type=jnp.float32)
    a0 = jnp.dot(e0, wa_ref[0], preferred_element_type=jnp.float32)
    a1 = jnp.dot(e1, wa_ref[1], preferred_element_type=jnp.float32)
    m = jnp.maximum(a0, a1)
    x0 = jnp.exp(a0 - m)
    x1 = jnp.exp(a1 - m)
    inv = 1.0 / (x0 + x1)
    z_ref[...] = (x0 * inv) * e0 + (x1 * inv) * e1


def _decoder_body(z_ref, zt_ref, o_ref):
    o_ref[...] = jnp.dot(z_ref[...], zt_ref[...],
                         preferred_element_type=jnp.float32)


def kernel(features, edge_index0, edge_index1,
           W1_0, W2_0, Watt_0, W1_1, W2_1, Watt_1):
    f32 = jnp.float32

    nk = jax.random.key(42)
    noise = jnp.stack([
        jax.random.normal(jax.random.fold_in(nk, 0), (_N, _H1), dtype=f32),
        jax.random.normal(jax.random.fold_in(nk, 1), (_N, _H1), dtype=f32),
    ]) * 0.1

    src = jnp.stack([edge_index0[0], edge_index1[0] + _N])
    src = src.reshape(2, _NT * _NCH, _CH)
    dst = jnp.stack([edge_index0[1], edge_index1[1]])
    dst = dst.reshape(2, _NT * _NCH, _CH)

    w1 = jnp.stack([W1_0, W1_1])
    w2 = jnp.stack([W2_0, W2_1])
    wa = jnp.stack([Watt_0, Watt_1])

    xw1 = pl.pallas_call(
        _xw_body,
        grid=(2,),
        in_specs=[
            pl.BlockSpec((_N, _D), lambda h: (0, 0)),
            pl.BlockSpec((1, _D, _H1), lambda h: (h, 0, 0)),
        ],
        out_specs=pl.BlockSpec((1, _N, _H1), lambda h: (h, 0, 0)),
        out_shape=jax.ShapeDtypeStruct((2, _N, _H1), f32),
    )(features, w1)

    l1 = _segment_sum_sc(src, dst, xw1.reshape(2 * _N, _H1))[:, :_N]

    noisy = pl.pallas_call(
        _relu_noise_body,
        out_shape=jax.ShapeDtypeStruct((2, _N, _H1), f32),
    )(l1, noise)

    s2 = _segment_sum_sc(src, dst, noisy.reshape(2 * _N, _H1))[:, :_N]

    z = pl.pallas_call(
        _combine_body,
        out_shape=jax.ShapeDtypeStruct((_N, _H2), f32),
    )(s2, w2, wa)

    recon = pl.pallas_call(
        _decoder_body,
        grid=(_N // _BM, pl.cdiv(_N, _BN)),
        in_specs=[
            pl.BlockSpec((_BM, _H2), lambda i, j: (i, 0)),
            pl.BlockSpec((_H2, _BN), lambda i, j: (0, j)),
        ],
        out_specs=pl.BlockSpec((_BM, _BN), lambda i, j: (i, j)),
        out_shape=jax.ShapeDtypeStruct((_N, _N), f32),
    )(z, z.T)

    return recon.reshape(-1)


# speedup vs baseline: 1.1182x; 1.1182x over previous
import jax
import jax.numpy as jnp
from jax import lax
from jax.experimental import pallas as pl
from jax.experimental.pallas import tpu as pltpu
from jax.experimental.pallas import tpu_sc as plsc

_N = 10000
_D = 128
_H1 = 32
_H2 = 16
_E = 160000

_CH = 125
_NT = 16
_EPT = _E // _NT
_NCH = _EPT // _CH
_RPT = 632
_NPAD = _RPT * _NT

_BM = 400


def _make_seg_sum_body(width):
    lanes_per_row = width // 16

    def body(src_hbm, dst_hbm, tab_hbm, out_hbm,
             sidx, didx, rows, zbuf, acc, sem):
        c = lax.axis_index("c")
        s = lax.axis_index("s")

        def zstep(i, carry):
            for k in range(lanes_per_row):
                zbuf[i, pl.ds(16 * k, 16)] = jnp.zeros((16,), jnp.float32)
            return carry

        lax.fori_loop(0, _RPT, zstep, 0)
        pltpu.sync_copy(zbuf, acc.at[pl.ds(s * _RPT, _RPT)])

        pltpu.sync_copy(src_hbm.at[c, pl.ds(s * _NCH, _NCH)], sidx)
        pltpu.sync_copy(dst_hbm.at[c, pl.ds(s * _NCH, _NCH)], didx)
        plsc.subcore_barrier()

        pltpu.async_copy(tab_hbm.at[sidx.at[0]], rows.at[0], sem.at[0])

        def estep(j, carry):
            slot = lax.rem(j, 2)
            nxt = lax.rem(j + 1, 2)

            @pl.when(j + 1 < _NCH)
            def _prefetch():
                pltpu.async_copy(tab_hbm.at[sidx.at[j + 1]], rows.at[nxt],
                                 sem.at[nxt])

            pltpu.make_async_copy(tab_hbm.at[sidx.at[j]], rows.at[slot],
                                  sem.at[slot]).wait()
            pltpu.sync_copy(rows.at[slot], acc.at[didx.at[j]], add=True)
            return carry

        lax.fori_loop(0, _NCH, estep, 0)

        plsc.subcore_barrier()
        pltpu.sync_copy(acc.at[pl.ds(s * _RPT, _RPT)],
                        out_hbm.at[c, pl.ds(s * _RPT, _RPT)])

    return body


_seg_sum_kernel_cache = {}


def _segment_sum_sc(src, dst, tab, width):
    if width not in _seg_sum_kernel_cache:
        mesh = plsc.VectorSubcoreMesh(core_axis_name="c", subcore_axis_name="s",
                                      num_cores=2, num_subcores=_NT)
        k = pl.kernel(
            _make_seg_sum_body(width),
            out_type=jax.ShapeDtypeStruct((2, _NPAD, width), jnp.float32),
            mesh=mesh,
            scratch_types=[
                pltpu.VMEM((_NCH, _CH), jnp.int32),
                pltpu.VMEM((_NCH, _CH), jnp.int32),
                pltpu.VMEM((2, _CH, width), jnp.float32),
                pltpu.VMEM((_RPT, width), jnp.float32),
                pltpu.VMEM_SHARED((_NPAD, width), jnp.float32),
                pltpu.SemaphoreType.DMA((2,)),
            ],
            compiler_params=pltpu.CompilerParams(use_tc_tiling_on_sc=False),
        )
        _seg_sum_kernel_cache[width] = k
    return _seg_sum_kernel_cache[width](src, dst, tab)


def _xw_body(x_ref, w_ref, o_ref):
    o_ref[0] = jnp.dot(x_ref[...], w_ref[0],
                       preferred_element_type=jnp.float32)


def _h2_body(l1_ref, nz_ref, w2_ref, o_ref):
    noisy = jnp.maximum(l1_ref[0], 0.0) + nz_ref[0]
    o_ref[0] = jnp.dot(noisy, w2_ref[0], preferred_element_type=jnp.float32)


def _combine_body(emb_ref, wa_ref, z_ref):
    e0 = emb_ref[0]
    e1 = emb_ref[1]
    a0 = jnp.dot(e0, wa_ref[0], preferred_element_type=jnp.float32)
    a1 = jnp.dot(e1, wa_ref[1], preferred_element_type=jnp.float32)
    x0 = jnp.exp(a0)
    x1 = jnp.exp(a1)
    s = x0 + x1
    z_ref[...] = (x0 / s) * e0 + (x1 / s) * e1


def _decoder_body(z_ref, zt_ref, o_ref):
    o_ref[...] = jnp.dot(z_ref[...], zt_ref[...],
                         preferred_element_type=jnp.float32)


def kernel(features, edge_index0, edge_index1,
           W1_0, W2_0, Watt_0, W1_1, W2_1, Watt_1):
    f32 = jnp.float32

    nk = jax.random.key(42)
    noise = jnp.stack([
        jax.random.normal(jax.random.fold_in(nk, 0), (_N, _H1), dtype=f32),
        jax.random.normal(jax.random.fold_in(nk, 1), (_N, _H1), dtype=f32),
    ]) * 0.1

    src = jnp.stack([edge_index0[0], edge_index1[0] + _N])
    src = src.reshape(2, _NT * _NCH, _CH)
    dst = jnp.stack([edge_index0[1], edge_index1[1]])
    dst = dst.reshape(2, _NT * _NCH, _CH)

    w1 = jnp.stack([W1_0, W1_1])
    w2 = jnp.stack([W2_0, W2_1])
    wa = jnp.stack([Watt_0, Watt_1])

    xw1 = pl.pallas_call(
        _xw_body,
        grid=(2,),
        in_specs=[
            pl.BlockSpec((_N, _D), lambda h: (0, 0)),
            pl.BlockSpec((1, _D, _H1), lambda h: (h, 0, 0)),
        ],
        out_specs=pl.BlockSpec((1, _N, _H1), lambda h: (h, 0, 0)),
        out_shape=jax.ShapeDtypeStruct((2, _N, _H1), f32),
    )(features, w1)

    l1 = _segment_sum_sc(src, dst, xw1.reshape(2 * _N, _H1), _H1)[:, :_N]

    h2 = pl.pallas_call(
        _h2_body,
        grid=(2,),
        in_specs=[
            pl.BlockSpec((1, _N, _H1), lambda h: (h, 0, 0)),
            pl.BlockSpec((1, _N, _H1), lambda h: (h, 0, 0)),
            pl.BlockSpec((1, _H1, _H2), lambda h: (h, 0, 0)),
        ],
        out_specs=pl.BlockSpec((1, _N, _H2), lambda h: (h, 0, 0)),
        out_shape=jax.ShapeDtypeStruct((2, _N, _H2), f32),
    )(l1, noise, w2)

    emb = _segment_sum_sc(src, dst, h2.reshape(2 * _N, _H2), _H2)[:, :_N]

    z = pl.pallas_call(
        _combine_body,
        out_shape=jax.ShapeDtypeStruct((_N, _H2), f32),
    )(emb, wa)

    recon = pl.pallas_call(
        _decoder_body,
        grid=(_N // _BM,),
        in_specs=[
            pl.BlockSpec((_BM, _H2), lambda i: (i, 0)),
            pl.BlockSpec((_H2, _N), lambda i: (0, 0)),
        ],
        out_specs=pl.BlockSpec((_BM, _N), lambda i: (i, 0)),
        out_shape=jax.ShapeDtypeStruct((_N, _N), f32),
    )(z, z.T)

    return recon.reshape(-1)
